# baseline (speedup 1.0000x reference)
import functools
import jax
import jax.numpy as jnp
from jax import lax
from jax.experimental import pallas as pl
from jax.experimental.pallas import tpu as pltpu
from jax.experimental.pallas import tpu_sc as plsc

CB_SIZE = 1024
CB_DIM = 256
NTOK = 4608
NW = 32
TPW = NTOK // NW
GCH = TPW // 2


def _dist_kernel(x_ref, cb_ref, idx_ref):
    xb = x_ref[...]
    cb = cb_ref[...]
    mm = jax.lax.dot_general(xb, cb, (((1,), (1,)), ((), ())),
                             preferred_element_type=jnp.float32)
    xnorm = jnp.sum(xb * xb, axis=1, keepdims=True)
    cnorm = jnp.sum(cb * cb, axis=1, keepdims=True).T
    t = xnorm - 2.0 * mm + cnorm
    tb = xb.shape[0]
    m = jnp.min(t, axis=1, keepdims=True)
    iota = jax.lax.broadcasted_iota(jnp.int32, (tb, CB_SIZE), 1)
    idx = jnp.min(jnp.where(t == m, iota, CB_SIZE), axis=1)
    idx_ref[...] = idx.reshape(1, 1, tb)


def _loss_kernel(x_ref, q_ref, idx_ref, loss_ref, perp_ref, counts_ref):
    i = pl.program_id(0)
    n = pl.num_programs(0)
    xb = x_ref[...]
    quant = q_ref[...]
    d = quant - xb
    loss_ref[...] = d * d
    tb = xb.shape[0]
    idx = idx_ref[...].reshape(tb)
    iota = jax.lax.broadcasted_iota(jnp.int32, (tb, CB_SIZE), 1)
    onehot = (iota == idx[:, None]).astype(jnp.float32)
    ones_row = jnp.ones((1, tb), jnp.float32)
    part = jax.lax.dot_general(ones_row, onehot, (((1,), (0,)), ((), ())),
                               preferred_element_type=jnp.float32)

    @pl.when(i == 0)
    def _init():
        counts_ref[...] = part

    @pl.when(i > 0)
    def _acc():
        counts_ref[...] = counts_ref[...] + part

    @pl.when(i == n - 1)
    def _fin():
        prob = counts_ref[...] / jnp.float32(NTOK)
        ent = jnp.sum(prob * jnp.log(prob + 1e-10))
        perp_ref[...] = jnp.exp(-ent).reshape(1, 1)


def _make_sc_gather():
    mesh = plsc.VectorSubcoreMesh(core_axis_name="c", subcore_axis_name="s")

    @functools.partial(
        pl.kernel, mesh=mesh,
        out_type=jax.ShapeDtypeStruct((NTOK, CB_DIM), jnp.float32),
        scratch_types=[
            pltpu.VMEM((TPW,), jnp.int32),
            pltpu.VMEM((GCH, CB_DIM), jnp.float32),
            pltpu.VMEM((GCH, CB_DIM), jnp.float32),
            pltpu.SemaphoreType.DMA,
            pltpu.SemaphoreType.DMA,
        ],
    )
    def sc_gather(idx_hbm, cb_hbm, q_hbm, idx_v, rows0_v, rows1_v, sem0, sem1):
        c = lax.axis_index("c")
        s = lax.axis_index("s")
        wid = s * 2 + c
        base = wid * TPW
        pltpu.sync_copy(idx_hbm.at[pl.ds(base, TPW)], idx_v)
        cp0 = pltpu.async_copy(cb_hbm.at[idx_v.at[pl.ds(0, GCH)]], rows0_v, sem0)
        cp1 = pltpu.async_copy(cb_hbm.at[idx_v.at[pl.ds(GCH, GCH)]], rows1_v, sem1)
        cp0.wait()
        pltpu.sync_copy(rows0_v, q_hbm.at[pl.ds(base, GCH)])
        cp1.wait()
        pltpu.sync_copy(rows1_v, q_hbm.at[pl.ds(base + GCH, GCH)])

    return sc_gather


_sc_gather = _make_sc_gather()


def kernel(x, codebook):
    shape = x.shape
    flat = x.reshape(-1, shape[-1])
    ntok = flat.shape[0]
    nb = 4
    tb = ntok // nb

    idx3 = pl.pallas_call(
        _dist_kernel,
        grid=(nb,),
        in_specs=[
            pl.BlockSpec((tb, CB_DIM), lambda i: (i, 0)),
            pl.BlockSpec((CB_SIZE, CB_DIM), lambda i: (0, 0)),
        ],
        out_specs=pl.BlockSpec((1, 1, tb), lambda i: (i, 0, 0)),
        out_shape=jax.ShapeDtypeStruct((nb, 1, tb), jnp.int32),
    )(flat, codebook)

    idx_flat = idx3.reshape(ntok)
    quant = _sc_gather(idx_flat, codebook)

    loss, perp = pl.pallas_call(
        _loss_kernel,
        grid=(nb,),
        in_specs=[
            pl.BlockSpec((tb, CB_DIM), lambda i: (i, 0)),
            pl.BlockSpec((tb, CB_DIM), lambda i: (i, 0)),
            pl.BlockSpec((1, 1, tb), lambda i: (i, 0, 0)),
        ],
        out_specs=[
            pl.BlockSpec((tb, CB_DIM), lambda i: (i, 0)),
            pl.BlockSpec((1, 1), lambda i: (0, 0)),
        ],
        out_shape=[
            jax.ShapeDtypeStruct((ntok, CB_DIM), jnp.float32),
            jax.ShapeDtypeStruct((1, 1), jnp.float32),
        ],
        scratch_shapes=[pltpu.VMEM((1, CB_SIZE), jnp.float32)],
    )(flat, quant, idx3)

    return (quant.reshape(shape), loss.reshape(shape), perp[0, 0])


# speedup vs baseline: 2.5471x; 2.5471x over previous
import jax
import jax.numpy as jnp
from jax.experimental import pallas as pl
from jax.experimental.pallas import tpu as pltpu

CB_SIZE = 1024
CB_DIM = 256


def _vq_kernel(x_ref, cb_ref, q_ref, loss_ref, perp_ref, counts_ref, cn_ref):
    i = pl.program_id(0)
    n = pl.num_programs(0)
    xb = x_ref[...]
    cb = cb_ref[...]

    del cn_ref
    mm = jax.lax.dot_general(xb, cb, (((1,), (1,)), ((), ())),
                             preferred_element_type=jnp.float32)
    xnorm = jnp.sum(xb * xb, axis=1, keepdims=True)
    cnorm = jnp.sum(cb * cb, axis=1, keepdims=True).T
    t = xnorm - 2.0 * mm + cnorm
    tb = xb.shape[0]
    m = jnp.min(t, axis=1, keepdims=True)
    iota = jax.lax.broadcasted_iota(jnp.int32, (tb, CB_SIZE), 1)
    idx = jnp.min(jnp.where(t == m, iota, CB_SIZE), axis=1)
    onehot = (iota == idx[:, None]).astype(jnp.float32)
    quant = jax.lax.dot_general(onehot, cb, (((1,), (0,)), ((), ())),
                                preferred_element_type=jnp.float32)
    q_ref[...] = quant
    d = quant - xb
    loss_ref[...] = d * d
    ones_row = jnp.ones((1, tb), jnp.float32)
    part = jax.lax.dot_general(ones_row, onehot, (((1,), (0,)), ((), ())),
                               preferred_element_type=jnp.float32)

    @pl.when(i == 0)
    def _init():
        counts_ref[...] = part

    @pl.when(i > 0)
    def _acc():
        counts_ref[...] = counts_ref[...] + part

    @pl.when(i == n - 1)
    def _fin():
        total = jnp.float32(tb) * n
        prob = counts_ref[...] / total
        ent = jnp.sum(prob * jnp.log(prob + 1e-10))
        perp_ref[...] = jnp.exp(-ent).reshape(1, 1)


def kernel(x, codebook):
    shape = x.shape
    flat = x.reshape(-1, shape[-1])
    ntok = flat.shape[0]
    nb = 4
    tb = ntok // nb

    quant, loss, perp = pl.pallas_call(
        _vq_kernel,
        grid=(nb,),
        in_specs=[
            pl.BlockSpec((tb, CB_DIM), lambda i: (i, 0)),
            pl.BlockSpec((CB_SIZE, CB_DIM), lambda i: (0, 0)),
        ],
        out_specs=[
            pl.BlockSpec((tb, CB_DIM), lambda i: (i, 0)),
            pl.BlockSpec((tb, CB_DIM), lambda i: (i, 0)),
            pl.BlockSpec((1, 1), lambda i: (0, 0)),
        ],
        out_shape=[
            jax.ShapeDtypeStruct((ntok, CB_DIM), jnp.float32),
            jax.ShapeDtypeStruct((ntok, CB_DIM), jnp.float32),
            jax.ShapeDtypeStruct((1, 1), jnp.float32),
        ],
        scratch_shapes=[pltpu.VMEM((1, CB_SIZE), jnp.float32),
                        pltpu.VMEM((1, CB_SIZE), jnp.float32)],
    )(flat, codebook)

    return (quant.reshape(shape), loss.reshape(shape), perp[0, 0])
